# baseline (speedup 1.0000x reference)
import functools

import jax
import jax.numpy as jnp
from jax.experimental import pallas as pl
from jax.experimental.pallas import tpu as pltpu

_BLK = 8192
_GATHER_PAD = 256


def _fwd_body(n_ctx, vocab, n_blk,
              idx_ref, emb_ref, w1_ref, b1_ref, w2_ref, b2_ref,
              logits_ref, norm_ref, h_ref, acc_ref, gath_ref, sem):
    i = pl.program_id(0)

    @pl.when(i == 0)
    def _():
        @pl.loop(0, n_ctx)
        def _(j):
            pltpu.make_async_copy(
                emb_ref.at[idx_ref[0, j]], gath_ref.at[j], sem).start()

        @pl.loop(0, n_ctx)
        def _(j):
            pltpu.make_async_copy(
                emb_ref.at[idx_ref[0, j]], gath_ref.at[j], sem).wait()

        g = gath_ref[...]
        row = jax.lax.broadcasted_iota(jnp.int32, g.shape, 0)
        g = jnp.where(row < n_ctx, g, 0.0)
        embeds = jnp.sum(g, axis=0, keepdims=True)
        z = jax.lax.dot_general(embeds, w1_ref[...],
                                (((1,), (1,)), ((), ())),
                                preferred_element_type=jnp.float32)
        h_ref[...] = jnp.maximum(z + b1_ref[...], 0.0)
        acc_ref[0] = -jnp.inf
        acc_ref[1] = 0.0

    h = h_ref[...]
    raw = jax.lax.dot_general(h, w2_ref[...],
                              (((1,), (1,)), ((), ())),
                              preferred_element_type=jnp.float32)
    logits = raw.reshape(8, _BLK // 8) + b2_ref[...]
    logits_ref[...] = logits

    r = jax.lax.broadcasted_iota(jnp.int32, logits.shape, 0)
    c = jax.lax.broadcasted_iota(jnp.int32, logits.shape, 1)
    col = i * _BLK + r * (_BLK // 8) + c
    valid = col < vocab
    masked = jnp.where(valid, logits, -jnp.inf)
    m_old = acc_ref[0]
    m_new = jnp.maximum(m_old, jnp.max(masked))
    bsum = jnp.sum(jnp.where(valid, jnp.exp(logits - m_new), 0.0))
    acc_ref[1] = acc_ref[1] * jnp.exp(m_old - m_new) + bsum
    acc_ref[0] = m_new

    @pl.when(i == n_blk - 1)
    def _():
        norm_ref[0, 0] = acc_ref[0] + jnp.log(acc_ref[1])


def _sub_body(l_ref, norm_ref, o_ref):
    o_ref[...] = l_ref[...] - norm_ref[0, 0]


def kernel(inputs, emb, W1, b1, W2, b2):
    vocab, hidden = W2.shape
    n_ctx = inputs.shape[0]
    n_blk = pl.cdiv(vocab, _BLK)
    npad = n_blk * _BLK
    embed_dim = emb.shape[1]

    idx = jnp.pad(inputs.astype(jnp.int32),
                  (0, _GATHER_PAD - n_ctx)).reshape(1, _GATHER_PAD)
    b1r = b1.reshape(1, -1)
    b2d = jnp.pad(b2, (0, npad - vocab)).reshape(n_blk * 8, _BLK // 8)

    logits, norm = pl.pallas_call(
        functools.partial(_fwd_body, n_ctx, vocab, n_blk),
        grid=(n_blk,),
        in_specs=[
            pl.BlockSpec(memory_space=pltpu.SMEM),
            pl.BlockSpec(memory_space=pl.ANY),
            pl.BlockSpec((hidden, embed_dim), lambda i: (0, 0)),
            pl.BlockSpec((1, hidden), lambda i: (0, 0)),
            pl.BlockSpec((_BLK, hidden), lambda i: (i, 0)),
            pl.BlockSpec((8, _BLK // 8), lambda i: (i, 0)),
        ],
        out_specs=[
            pl.BlockSpec((8, _BLK // 8), lambda i: (i, 0)),
            pl.BlockSpec((1, 1), lambda i: (0, 0), memory_space=pltpu.SMEM),
        ],
        out_shape=[
            jax.ShapeDtypeStruct((n_blk * 8, _BLK // 8), jnp.float32),
            jax.ShapeDtypeStruct((1, 1), jnp.float32),
        ],
        scratch_shapes=[
            pltpu.VMEM((1, hidden), jnp.float32),
            pltpu.SMEM((2,), jnp.float32),
            pltpu.VMEM((_GATHER_PAD, embed_dim), jnp.float32),
            pltpu.SemaphoreType.DMA,
        ],
        compiler_params=pltpu.CompilerParams(
            dimension_semantics=("arbitrary",)),
    )(idx, emb, W1, b1r, W2, b2d)

    shifted = pl.pallas_call(
        _sub_body,
        grid=(n_blk,),
        in_specs=[
            pl.BlockSpec((8, _BLK // 8), lambda i: (i, 0)),
            pl.BlockSpec((1, 1), lambda i: (0, 0), memory_space=pltpu.SMEM),
        ],
        out_specs=pl.BlockSpec((8, _BLK // 8), lambda i: (i, 0)),
        out_shape=jax.ShapeDtypeStruct((n_blk * 8, _BLK // 8), jnp.float32),
    )(logits, norm)

    return shifted.reshape(npad)[:vocab].reshape(1, vocab)


# speedup vs baseline: 1.6727x; 1.6727x over previous
import functools

import jax
import jax.numpy as jnp
from jax.experimental import pallas as pl
from jax.experimental.pallas import tpu as pltpu

_BLK = 8192
_GATHER_PAD = 256


def _gather_body(n_ctx, embed_dim,
                 idx_ref, embt_ref, w1_ref, b1_ref, h_out_ref, acc_ref):
    j = pl.program_id(0)

    @pl.when(j == 0)
    def _():
        acc_ref[...] = jnp.zeros_like(acc_ref)

    lane = idx_ref[j] % 128
    blk = embt_ref[...]
    sel = jax.lax.broadcasted_iota(jnp.int32, blk.shape, 1) == lane
    acc_ref[...] += jnp.where(sel, blk, 0.0)

    @pl.when(j == n_ctx - 1)
    def _():
        embeds = jnp.sum(acc_ref[...], axis=1).reshape(1, embed_dim)
        z = jax.lax.dot_general(embeds, w1_ref[...],
                                (((1,), (1,)), ((), ())),
                                preferred_element_type=jnp.float32)
        h_out_ref[...] = jnp.maximum(z + b1_ref[...], 0.0)


def _fwd_body(vocab, n_blk,
              h_in_ref, w2_ref, b2_ref,
              logits_ref, norm_ref, acc_ref):
    i = pl.program_id(0)

    @pl.when(i == 0)
    def _():
        acc_ref[0] = -jnp.inf
        acc_ref[1] = 0.0

    h = h_in_ref[...]
    raw = jax.lax.dot_general(h, w2_ref[...],
                              (((1,), (1,)), ((), ())),
                              preferred_element_type=jnp.float32)
    logits = raw.reshape(8, _BLK // 8) + b2_ref[...]
    logits_ref[...] = logits

    r = jax.lax.broadcasted_iota(jnp.int32, logits.shape, 0)
    c = jax.lax.broadcasted_iota(jnp.int32, logits.shape, 1)
    col = i * _BLK + r * (_BLK // 8) + c
    valid = col < vocab
    masked = jnp.where(valid, logits, -jnp.inf)
    m_old = acc_ref[0]
    m_new = jnp.maximum(m_old, jnp.max(masked))
    bsum = jnp.sum(jnp.where(valid, jnp.exp(logits - m_new), 0.0))
    acc_ref[1] = acc_ref[1] * jnp.exp(m_old - m_new) + bsum
    acc_ref[0] = m_new

    @pl.when(i == n_blk - 1)
    def _():
        norm_ref[0, 0] = acc_ref[0] + jnp.log(acc_ref[1])


def _sub_body(l_ref, norm_ref, o_ref):
    o_ref[...] = l_ref[...] - norm_ref[0, 0]


def kernel(inputs, emb, W1, b1, W2, b2):
    vocab, hidden = W2.shape
    n_ctx = inputs.shape[0]
    n_blk = pl.cdiv(vocab, _BLK)
    npad = n_blk * _BLK
    embed_dim = emb.shape[1]

    idx = inputs.astype(jnp.int32)
    b1r = b1.reshape(1, -1)
    b2d = jnp.pad(b2, (0, npad - vocab)).reshape(n_blk * 8, _BLK // 8)
    embt = jnp.swapaxes(emb, 0, 1)

    h = pl.pallas_call(
        functools.partial(_gather_body, n_ctx, embed_dim),
        grid_spec=pltpu.PrefetchScalarGridSpec(
            num_scalar_prefetch=1,
            grid=(n_ctx,),
            in_specs=[
                pl.BlockSpec((embed_dim, 128),
                             lambda j, idx_ref: (0, idx_ref[j] // 128)),
                pl.BlockSpec((hidden, embed_dim), lambda j, idx_ref: (0, 0)),
                pl.BlockSpec((1, hidden), lambda j, idx_ref: (0, 0)),
            ],
            out_specs=pl.BlockSpec((1, hidden), lambda j, idx_ref: (0, 0)),
            scratch_shapes=[
                pltpu.VMEM((embed_dim, 128), jnp.float32),
            ],
        ),
        out_shape=jax.ShapeDtypeStruct((1, hidden), jnp.float32),
        compiler_params=pltpu.CompilerParams(
            dimension_semantics=("arbitrary",)),
    )(idx, embt, W1, b1r)

    logits, norm = pl.pallas_call(
        functools.partial(_fwd_body, vocab, n_blk),
        grid=(n_blk,),
        in_specs=[
            pl.BlockSpec((1, hidden), lambda i: (0, 0)),
            pl.BlockSpec((_BLK, hidden), lambda i: (i, 0)),
            pl.BlockSpec((8, _BLK // 8), lambda i: (i, 0)),
        ],
        out_specs=[
            pl.BlockSpec((8, _BLK // 8), lambda i: (i, 0)),
            pl.BlockSpec((1, 1), lambda i: (0, 0), memory_space=pltpu.SMEM),
        ],
        out_shape=[
            jax.ShapeDtypeStruct((n_blk * 8, _BLK // 8), jnp.float32),
            jax.ShapeDtypeStruct((1, 1), jnp.float32),
        ],
        scratch_shapes=[
            pltpu.SMEM((2,), jnp.float32),
        ],
        compiler_params=pltpu.CompilerParams(
            dimension_semantics=("arbitrary",)),
    )(h, W2, b2d)

    shifted = pl.pallas_call(
        _sub_body,
        grid=(n_blk,),
        in_specs=[
            pl.BlockSpec((8, _BLK // 8), lambda i: (i, 0)),
            pl.BlockSpec((1, 1), lambda i: (0, 0), memory_space=pltpu.SMEM),
        ],
        out_specs=pl.BlockSpec((8, _BLK // 8), lambda i: (i, 0)),
        out_shape=jax.ShapeDtypeStruct((n_blk * 8, _BLK // 8), jnp.float32),
    )(logits, norm)

    return shifted.reshape(npad)[:vocab].reshape(1, vocab)


# speedup vs baseline: 2.4992x; 1.4941x over previous
import functools

import jax
import jax.numpy as jnp
from jax.experimental import pallas as pl
from jax.experimental.pallas import tpu as pltpu

_BLK = 8192
_GATHER_PAD = 256


def _gather_body(n_ctx, n_par, embed_dim, idx_ref, *refs):
    embt_refs = refs[:n_par]
    w1_ref, b1_ref, h_out_ref, acc_ref = refs[n_par:]
    j = pl.program_id(0)

    @pl.when(j == 0)
    def _():
        acc_ref[...] = jnp.zeros_like(acc_ref)

    total = acc_ref[...]
    for k in range(n_par):
        lane = idx_ref[n_par * j + k] % 128
        blk = embt_refs[k][...]
        sel = jax.lax.broadcasted_iota(jnp.int32, blk.shape, 1) == lane
        total += jnp.where(sel, blk, 0.0)
    acc_ref[...] = total

    @pl.when(j == n_ctx // n_par - 1)
    def _():
        embeds = jnp.sum(acc_ref[...], axis=1).reshape(1, embed_dim)
        z = jax.lax.dot_general(embeds, w1_ref[...],
                                (((1,), (1,)), ((), ())),
                                preferred_element_type=jnp.float32)
        h_out_ref[...] = jnp.maximum(z + b1_ref[...], 0.0)


def _fwd_body(vocab, n_blk,
              h_in_ref, w2_ref, b2_ref,
              logits_ref, norm_ref, acc_ref):
    i = pl.program_id(0)

    @pl.when(i == 0)
    def _():
        acc_ref[0] = -jnp.inf
        acc_ref[1] = 0.0

    h = h_in_ref[...]
    raw = jax.lax.dot_general(h, w2_ref[...],
                              (((1,), (1,)), ((), ())),
                              preferred_element_type=jnp.float32)
    logits = raw.reshape(8, _BLK // 8) + b2_ref[...]
    logits_ref[...] = logits

    r = jax.lax.broadcasted_iota(jnp.int32, logits.shape, 0)
    c = jax.lax.broadcasted_iota(jnp.int32, logits.shape, 1)
    col = i * _BLK + r * (_BLK // 8) + c
    valid = col < vocab
    masked = jnp.where(valid, logits, -jnp.inf)
    m_old = acc_ref[0]
    m_new = jnp.maximum(m_old, jnp.max(masked))
    bsum = jnp.sum(jnp.where(valid, jnp.exp(logits - m_new), 0.0))
    acc_ref[1] = acc_ref[1] * jnp.exp(m_old - m_new) + bsum
    acc_ref[0] = m_new

    @pl.when(i == n_blk - 1)
    def _():
        norm_ref[0, 0] = acc_ref[0] + jnp.log(acc_ref[1])


def _sub_body(l_ref, norm_ref, o_ref):
    o_ref[...] = l_ref[...] - norm_ref[0, 0]


def kernel(inputs, emb, W1, b1, W2, b2):
    vocab, hidden = W2.shape
    n_ctx = inputs.shape[0]
    n_blk = pl.cdiv(vocab, _BLK)
    npad = n_blk * _BLK
    embed_dim = emb.shape[1]

    idx = inputs.astype(jnp.int32)
    b1r = b1.reshape(1, -1)
    b2d = jnp.pad(b2, (0, npad - vocab)).reshape(n_blk * 8, _BLK // 8)
    embt = jnp.swapaxes(emb, 0, 1)

    n_par = 8
    while n_ctx % n_par:
        n_par //= 2

    def _embt_spec(k):
        return pl.BlockSpec(
            (embed_dim, 128),
            lambda j, idx_ref: (0, idx_ref[n_par * j + k] // 128))

    h = pl.pallas_call(
        functools.partial(_gather_body, n_ctx, n_par, embed_dim),
        grid_spec=pltpu.PrefetchScalarGridSpec(
            num_scalar_prefetch=1,
            grid=(n_ctx // n_par,),
            in_specs=[_embt_spec(k) for k in range(n_par)] + [
                pl.BlockSpec((hidden, embed_dim), lambda j, idx_ref: (0, 0)),
                pl.BlockSpec((1, hidden), lambda j, idx_ref: (0, 0)),
            ],
            out_specs=pl.BlockSpec((1, hidden), lambda j, idx_ref: (0, 0)),
            scratch_shapes=[
                pltpu.VMEM((embed_dim, 128), jnp.float32),
            ],
        ),
        out_shape=jax.ShapeDtypeStruct((1, hidden), jnp.float32),
        compiler_params=pltpu.CompilerParams(
            dimension_semantics=("arbitrary",)),
    )(idx, *([embt] * n_par), W1, b1r)

    logits, norm = pl.pallas_call(
        functools.partial(_fwd_body, vocab, n_blk),
        grid=(n_blk,),
        in_specs=[
            pl.BlockSpec((1, hidden), lambda i: (0, 0)),
            pl.BlockSpec((_BLK, hidden), lambda i: (i, 0)),
            pl.BlockSpec((8, _BLK // 8), lambda i: (i, 0)),
        ],
        out_specs=[
            pl.BlockSpec((8, _BLK // 8), lambda i: (i, 0)),
            pl.BlockSpec((1, 1), lambda i: (0, 0), memory_space=pltpu.SMEM),
        ],
        out_shape=[
            jax.ShapeDtypeStruct((n_blk * 8, _BLK // 8), jnp.float32),
            jax.ShapeDtypeStruct((1, 1), jnp.float32),
        ],
        scratch_shapes=[
            pltpu.SMEM((2,), jnp.float32),
        ],
        compiler_params=pltpu.CompilerParams(
            dimension_semantics=("arbitrary",)),
    )(h, W2, b2d)

    shifted = pl.pallas_call(
        _sub_body,
        in_specs=[
            pl.BlockSpec((n_blk * 8, _BLK // 8), lambda: (0, 0)),
            pl.BlockSpec((1, 1), lambda: (0, 0), memory_space=pltpu.SMEM),
        ],
        out_specs=pl.BlockSpec((n_blk * 8, _BLK // 8), lambda: (0, 0)),
        out_shape=jax.ShapeDtypeStruct((n_blk * 8, _BLK // 8), jnp.float32),
    )(logits, norm)

    return shifted.reshape(npad)[:vocab].reshape(1, vocab)


# speedup vs baseline: 3.0372x; 1.2153x over previous
import functools

import jax
import jax.numpy as jnp
from jax.experimental import pallas as pl
from jax.experimental.pallas import tpu as pltpu

_BLK = 16384
_GATHER_PAD = 256


def _gather_body(n_ctx, n_par, embed_dim, idx_ref, *refs):
    embt_refs = refs[:n_par]
    w1_ref, b1_ref, h_out_ref, acc_ref = refs[n_par:]
    j = pl.program_id(0)

    @pl.when(j == 0)
    def _():
        acc_ref[...] = jnp.zeros_like(acc_ref)

    total = acc_ref[...]
    for k in range(n_par):
        lane = idx_ref[n_par * j + k] % 128
        blk = embt_refs[k][...]
        sel = jax.lax.broadcasted_iota(jnp.int32, blk.shape, 1) == lane
        total += jnp.where(sel, blk, 0.0)
    acc_ref[...] = total

    @pl.when(j == n_ctx // n_par - 1)
    def _():
        embeds = jnp.sum(acc_ref[...], axis=1).reshape(1, embed_dim)
        z = jax.lax.dot_general(embeds, w1_ref[...],
                                (((1,), (1,)), ((), ())),
                                preferred_element_type=jnp.float32)
        h_out_ref[...] = jnp.maximum(z + b1_ref[...], 0.0)


def _fwd_body(vocab, n_blk,
              h_in_ref, w2_ref, b2_ref,
              logits_ref, norm_ref, acc_ref):
    i = pl.program_id(0)

    @pl.when(i == 0)
    def _():
        acc_ref[0] = -jnp.inf
        acc_ref[1] = 0.0

    h = h_in_ref[...]
    raw = jax.lax.dot_general(h, w2_ref[...],
                              (((1,), (1,)), ((), ())),
                              preferred_element_type=jnp.float32)
    logits = raw.reshape(8, _BLK // 8) + b2_ref[...]
    logits_ref[...] = logits

    r = jax.lax.broadcasted_iota(jnp.int32, logits.shape, 0)
    c = jax.lax.broadcasted_iota(jnp.int32, logits.shape, 1)
    col = i * _BLK + r * (_BLK // 8) + c
    valid = col < vocab
    masked = jnp.where(valid, logits, -jnp.inf)
    m_old = acc_ref[0]
    m_new = jnp.maximum(m_old, jnp.max(masked))
    bsum = jnp.sum(jnp.where(valid, jnp.exp(logits - m_new), 0.0))
    acc_ref[1] = acc_ref[1] * jnp.exp(m_old - m_new) + bsum
    acc_ref[0] = m_new

    @pl.when(i == n_blk - 1)
    def _():
        norm_ref[0, 0] = acc_ref[0] + jnp.log(acc_ref[1])


def _sub_body(l_ref, norm_ref, o_ref):
    o_ref[...] = l_ref[...] - norm_ref[0, 0]


def kernel(inputs, emb, W1, b1, W2, b2):
    vocab, hidden = W2.shape
    n_ctx = inputs.shape[0]
    n_blk = pl.cdiv(vocab, _BLK)
    npad = n_blk * _BLK
    embed_dim = emb.shape[1]

    idx = inputs.astype(jnp.int32)
    b1r = b1.reshape(1, -1)
    b2d = jnp.pad(b2, (0, npad - vocab)).reshape(n_blk * 8, _BLK // 8)
    embt = jnp.swapaxes(emb, 0, 1)

    n_par = 8
    while n_ctx % n_par:
        n_par //= 2

    def _embt_spec(k):
        return pl.BlockSpec(
            (embed_dim, 128),
            lambda j, idx_ref: (0, idx_ref[n_par * j + k] // 128))

    h = pl.pallas_call(
        functools.partial(_gather_body, n_ctx, n_par, embed_dim),
        grid_spec=pltpu.PrefetchScalarGridSpec(
            num_scalar_prefetch=1,
            grid=(n_ctx // n_par,),
            in_specs=[_embt_spec(k) for k in range(n_par)] + [
                pl.BlockSpec((hidden, embed_dim), lambda j, idx_ref: (0, 0)),
                pl.BlockSpec((1, hidden), lambda j, idx_ref: (0, 0)),
            ],
            out_specs=pl.BlockSpec((1, hidden), lambda j, idx_ref: (0, 0)),
            scratch_shapes=[
                pltpu.VMEM((embed_dim, 128), jnp.float32),
            ],
        ),
        out_shape=jax.ShapeDtypeStruct((1, hidden), jnp.float32),
        compiler_params=pltpu.CompilerParams(
            dimension_semantics=("arbitrary",)),
    )(idx, *([embt] * n_par), W1, b1r)

    logits, norm = pl.pallas_call(
        functools.partial(_fwd_body, vocab, n_blk),
        grid=(n_blk,),
        in_specs=[
            pl.BlockSpec((1, hidden), lambda i: (0, 0)),
            pl.BlockSpec((_BLK, hidden), lambda i: (i, 0)),
            pl.BlockSpec((8, _BLK // 8), lambda i: (i, 0)),
        ],
        out_specs=[
            pl.BlockSpec((8, _BLK // 8), lambda i: (i, 0)),
            pl.BlockSpec((1, 1), lambda i: (0, 0), memory_space=pltpu.SMEM),
        ],
        out_shape=[
            jax.ShapeDtypeStruct((n_blk * 8, _BLK // 8), jnp.float32),
            jax.ShapeDtypeStruct((1, 1), jnp.float32),
        ],
        scratch_shapes=[
            pltpu.SMEM((2,), jnp.float32),
        ],
        compiler_params=pltpu.CompilerParams(
            dimension_semantics=("arbitrary",)),
    )(h, W2, b2d)

    shifted = pl.pallas_call(
        _sub_body,
        in_specs=[
            pl.BlockSpec((n_blk * 8, _BLK // 8), lambda: (0, 0)),
            pl.BlockSpec((1, 1), lambda: (0, 0), memory_space=pltpu.SMEM),
        ],
        out_specs=pl.BlockSpec((n_blk * 8, _BLK // 8), lambda: (0, 0)),
        out_shape=jax.ShapeDtypeStruct((n_blk * 8, _BLK // 8), jnp.float32),
    )(logits, norm)

    return shifted.reshape(npad)[:vocab].reshape(1, vocab)


# speedup vs baseline: 3.2037x; 1.0548x over previous
import functools

import jax
import jax.numpy as jnp
from jax.experimental import pallas as pl
from jax.experimental.pallas import tpu as pltpu

_BLK = 32768
_GATHER_PAD = 256


def _gather_body(n_ctx, n_par, embed_dim, idx_ref, *refs):
    embt_refs = refs[:n_par]
    w1_ref, b1_ref, h_out_ref, acc_ref = refs[n_par:]
    j = pl.program_id(0)

    @pl.when(j == 0)
    def _():
        acc_ref[...] = jnp.zeros_like(acc_ref)

    total = acc_ref[...]
    for k in range(n_par):
        lane = idx_ref[n_par * j + k] % 128
        blk = embt_refs[k][...]
        sel = jax.lax.broadcasted_iota(jnp.int32, blk.shape, 1) == lane
        total += jnp.where(sel, blk, 0.0)
    acc_ref[...] = total

    @pl.when(j == n_ctx // n_par - 1)
    def _():
        embeds = jnp.sum(acc_ref[...], axis=1).reshape(1, embed_dim)
        z = jax.lax.dot_general(embeds, w1_ref[...],
                                (((1,), (1,)), ((), ())),
                                preferred_element_type=jnp.float32)
        h_out_ref[...] = jnp.maximum(z + b1_ref[...], 0.0)


def _fwd_body(vocab, n_blk,
              h_in_ref, w2_ref, b2_ref,
              logits_ref, norm_ref, acc_ref):
    i = pl.program_id(0)

    @pl.when(i == 0)
    def _():
        acc_ref[0] = -jnp.inf
        acc_ref[1] = 0.0

    h = h_in_ref[...]
    raw = jax.lax.dot_general(h, w2_ref[...],
                              (((1,), (1,)), ((), ())),
                              preferred_element_type=jnp.float32,
                              precision=jax.lax.Precision.DEFAULT)
    logits = raw.reshape(8, _BLK // 8) + b2_ref[...]
    logits_ref[...] = logits

    m_old = acc_ref[0]

    @pl.when(i < n_blk - 1)
    def _():
        m_new = jnp.maximum(m_old, jnp.max(logits))
        bsum = jnp.sum(jnp.exp(logits - m_new))
        acc_ref[1] = acc_ref[1] * jnp.exp(m_old - m_new) + bsum
        acc_ref[0] = m_new

    @pl.when(i == n_blk - 1)
    def _():
        r = jax.lax.broadcasted_iota(jnp.int32, logits.shape, 0)
        c = jax.lax.broadcasted_iota(jnp.int32, logits.shape, 1)
        col = i * _BLK + r * (_BLK // 8) + c
        valid = col < vocab
        masked = jnp.where(valid, logits, -jnp.inf)
        m_new = jnp.maximum(m_old, jnp.max(masked))
        bsum = jnp.sum(jnp.where(valid, jnp.exp(logits - m_new), 0.0))
        s = acc_ref[1] * jnp.exp(m_old - m_new) + bsum
        norm_ref[0, 0] = m_new + jnp.log(s)


def _sub_body(l_ref, norm_ref, o_ref):
    o_ref[...] = l_ref[...] - norm_ref[0, 0]


def kernel(inputs, emb, W1, b1, W2, b2):
    vocab, hidden = W2.shape
    n_ctx = inputs.shape[0]
    n_blk = pl.cdiv(vocab, _BLK)
    npad = n_blk * _BLK
    embed_dim = emb.shape[1]

    idx = inputs.astype(jnp.int32)
    b1r = b1.reshape(1, -1)
    b2d = jnp.pad(b2, (0, npad - vocab)).reshape(n_blk * 8, _BLK // 8)
    embt = jnp.swapaxes(emb, 0, 1)

    n_par = next(g for g in (25, 20, 16, 10, 8, 5, 4, 2, 1)
                 if n_ctx % g == 0)

    def _embt_spec(k):
        return pl.BlockSpec(
            (embed_dim, 128),
            lambda j, idx_ref: (0, idx_ref[n_par * j + k] // 128))

    h = pl.pallas_call(
        functools.partial(_gather_body, n_ctx, n_par, embed_dim),
        grid_spec=pltpu.PrefetchScalarGridSpec(
            num_scalar_prefetch=1,
            grid=(n_ctx // n_par,),
            in_specs=[_embt_spec(k) for k in range(n_par)] + [
                pl.BlockSpec((hidden, embed_dim), lambda j, idx_ref: (0, 0)),
                pl.BlockSpec((1, hidden), lambda j, idx_ref: (0, 0)),
            ],
            out_specs=pl.BlockSpec((1, hidden), lambda j, idx_ref: (0, 0)),
            scratch_shapes=[
                pltpu.VMEM((embed_dim, 128), jnp.float32),
            ],
        ),
        out_shape=jax.ShapeDtypeStruct((1, hidden), jnp.float32),
        compiler_params=pltpu.CompilerParams(
            dimension_semantics=("arbitrary",)),
    )(idx, *([embt] * n_par), W1, b1r)

    logits, norm = pl.pallas_call(
        functools.partial(_fwd_body, vocab, n_blk),
        grid=(n_blk,),
        in_specs=[
            pl.BlockSpec((1, hidden), lambda i: (0, 0)),
            pl.BlockSpec((_BLK, hidden), lambda i: (i, 0)),
            pl.BlockSpec((8, _BLK // 8), lambda i: (i, 0)),
        ],
        out_specs=[
            pl.BlockSpec((8, _BLK // 8), lambda i: (i, 0)),
            pl.BlockSpec((1, 1), lambda i: (0, 0), memory_space=pltpu.SMEM),
        ],
        out_shape=[
            jax.ShapeDtypeStruct((n_blk * 8, _BLK // 8), jnp.float32),
            jax.ShapeDtypeStruct((1, 1), jnp.float32),
        ],
        scratch_shapes=[
            pltpu.SMEM((2,), jnp.float32),
        ],
        compiler_params=pltpu.CompilerParams(
            dimension_semantics=("arbitrary",)),
    )(h, W2, b2d)

    shifted = pl.pallas_call(
        _sub_body,
        in_specs=[
            pl.BlockSpec((n_blk * 8, _BLK // 8), lambda: (0, 0)),
            pl.BlockSpec((1, 1), lambda: (0, 0), memory_space=pltpu.SMEM),
        ],
        out_specs=pl.BlockSpec((n_blk * 8, _BLK // 8), lambda: (0, 0)),
        out_shape=jax.ShapeDtypeStruct((n_blk * 8, _BLK // 8), jnp.float32),
    )(logits, norm)

    return shifted.reshape(npad)[:vocab].reshape(1, vocab)


# speedup vs baseline: 3.2768x; 1.0228x over previous
import functools

import jax
import jax.numpy as jnp
from jax.experimental import pallas as pl
from jax.experimental.pallas import tpu as pltpu

_BLK = 32768
_GATHER_PAD = 256


def _gather_body(n_ctx, n_par, embed_dim, idx_ref, *refs):
    embt_refs = refs[:n_par]
    w1_ref, b1_ref, h_out_ref, acc_ref = refs[n_par:]
    j = pl.program_id(0)

    @pl.when(j == 0)
    def _():
        acc_ref[...] = jnp.zeros_like(acc_ref)

    total = acc_ref[...]
    for k in range(n_par):
        lane = idx_ref[n_par * j + k] % 128
        blk = embt_refs[k][...]
        sel = jax.lax.broadcasted_iota(jnp.int32, blk.shape, 1) == lane
        total += jnp.where(sel, blk, 0.0)
    acc_ref[...] = total

    @pl.when(j == n_ctx // n_par - 1)
    def _():
        embeds = jnp.sum(acc_ref[...], axis=1).reshape(1, embed_dim)
        z = jax.lax.dot_general(embeds, w1_ref[...],
                                (((1,), (1,)), ((), ())),
                                preferred_element_type=jnp.float32)
        h_out_ref[...] = jnp.maximum(z + b1_ref[...], 0.0)


def _fwd_body(vocab, n_blk,
              h_in_ref, w2_ref, b2_ref,
              out_ref, acc_ref, lsc_ref):
    i = pl.program_id(0)

    @pl.when(i == 0)
    def _():
        acc_ref[0] = -jnp.inf
        acc_ref[1] = 0.0

    @pl.when(i < n_blk)
    def _():
        h = h_in_ref[...]
        raw = jax.lax.dot_general(h, w2_ref[...],
                                  (((1,), (1,)), ((), ())),
                                  preferred_element_type=jnp.float32,
                                  precision=jax.lax.Precision.DEFAULT)
        logits = raw.reshape(8, _BLK // 8) + b2_ref[...]
        lsc_ref[pl.ds(i * 8, 8), :] = logits

        m_old = acc_ref[0]

        @pl.when(i < n_blk - 1)
        def _():
            m_new = jnp.maximum(m_old, jnp.max(logits))
            bsum = jnp.sum(jnp.exp(logits - m_new))
            acc_ref[1] = acc_ref[1] * jnp.exp(m_old - m_new) + bsum
            acc_ref[0] = m_new

        @pl.when(i == n_blk - 1)
        def _():
            r = jax.lax.broadcasted_iota(jnp.int32, logits.shape, 0)
            c = jax.lax.broadcasted_iota(jnp.int32, logits.shape, 1)
            col = i * _BLK + r * (_BLK // 8) + c
            valid = col < vocab
            masked = jnp.where(valid, logits, -jnp.inf)
            m_new = jnp.maximum(m_old, jnp.max(masked))
            bsum = jnp.sum(jnp.where(valid, jnp.exp(logits - m_new), 0.0))
            s = acc_ref[1] * jnp.exp(m_old - m_new) + bsum
            acc_ref[0] = m_new + jnp.log(s)

    @pl.when(i == n_blk)
    def _():
        out_ref[...] = lsc_ref[...] - acc_ref[0]


def kernel(inputs, emb, W1, b1, W2, b2):
    vocab, hidden = W2.shape
    n_ctx = inputs.shape[0]
    n_blk = pl.cdiv(vocab, _BLK)
    npad = n_blk * _BLK
    embed_dim = emb.shape[1]

    idx = inputs.astype(jnp.int32)
    b1r = b1.reshape(1, -1)
    b2d = jnp.pad(b2, (0, npad - vocab)).reshape(n_blk * 8, _BLK // 8)
    embt = jnp.swapaxes(emb, 0, 1)

    n_par = next(g for g in (25, 20, 16, 10, 8, 5, 4, 2, 1)
                 if n_ctx % g == 0)

    def _embt_spec(k):
        return pl.BlockSpec(
            (embed_dim, 128),
            lambda j, idx_ref: (0, idx_ref[n_par * j + k] // 128))

    h = pl.pallas_call(
        functools.partial(_gather_body, n_ctx, n_par, embed_dim),
        grid_spec=pltpu.PrefetchScalarGridSpec(
            num_scalar_prefetch=1,
            grid=(n_ctx // n_par,),
            in_specs=[_embt_spec(k) for k in range(n_par)] + [
                pl.BlockSpec((hidden, embed_dim), lambda j, idx_ref: (0, 0)),
                pl.BlockSpec((1, hidden), lambda j, idx_ref: (0, 0)),
            ],
            out_specs=pl.BlockSpec((1, hidden), lambda j, idx_ref: (0, 0)),
            scratch_shapes=[
                pltpu.VMEM((embed_dim, 128), jnp.float32),
            ],
        ),
        out_shape=jax.ShapeDtypeStruct((1, hidden), jnp.float32),
        compiler_params=pltpu.CompilerParams(
            dimension_semantics=("arbitrary",)),
    )(idx, *([embt] * n_par), W1, b1r)

    last = n_blk - 1
    shifted = pl.pallas_call(
        functools.partial(_fwd_body, vocab, n_blk),
        grid=(n_blk + 1,),
        in_specs=[
            pl.BlockSpec((1, hidden), lambda i: (0, 0)),
            pl.BlockSpec((_BLK, hidden), lambda i: (jnp.minimum(i, last), 0)),
            pl.BlockSpec((8, _BLK // 8), lambda i: (jnp.minimum(i, last), 0)),
        ],
        out_specs=pl.BlockSpec((n_blk * 8, _BLK // 8), lambda i: (0, 0)),
        out_shape=jax.ShapeDtypeStruct((n_blk * 8, _BLK // 8), jnp.float32),
        scratch_shapes=[
            pltpu.SMEM((2,), jnp.float32),
            pltpu.VMEM((n_blk * 8, _BLK // 8), jnp.float32),
        ],
        compiler_params=pltpu.CompilerParams(
            dimension_semantics=("arbitrary",)),
    )(h, W2, b2d)

    return shifted.reshape(npad)[:vocab].reshape(1, vocab)
